# baseline (speedup 1.0000x reference)
import jax
import jax.numpy as jnp
from jax.experimental import pallas as pl


def _body(x_ref, xe_ref, ye_ref, o_ref):
    pos = xe_ref[...][:, None, :] + ye_ref[...][None, :, :]
    o_ref[...] = x_ref[...] + pos[None]


def kernel(x, xemb, yemb):
    B, H, W, D = x.shape
    BB = 8
    return pl.pallas_call(
        _body,
        grid=(B // BB,),
        in_specs=[
            pl.BlockSpec((BB, H, W, D), lambda i: (i, 0, 0, 0)),
            pl.BlockSpec((H, D), lambda i: (0, 0)),
            pl.BlockSpec((W, D), lambda i: (0, 0)),
        ],
        out_specs=pl.BlockSpec((BB, H, W, D), lambda i: (i, 0, 0, 0)),
        out_shape=jax.ShapeDtypeStruct(x.shape, x.dtype),
    )(x, xemb, yemb)


# speedup vs baseline: 1.3519x; 1.3519x over previous
import jax
import jax.numpy as jnp
from jax.experimental import pallas as pl


def _pos_body(xe_ref, ye_ref, pos_ref):
    pos_ref[...] = xe_ref[...][:, None, :] + ye_ref[...][None, :, :]


def _add_body(x_ref, pos_ref, o_ref):
    o_ref[...] = x_ref[...] + pos_ref[...][None]


def kernel(x, xemb, yemb):
    B, H, W, D = x.shape
    n = H * W * D
    LANES = 8192
    rows = n // LANES

    posemb = pl.pallas_call(
        _pos_body,
        out_shape=jax.ShapeDtypeStruct((H, W, D), x.dtype),
    )(xemb, yemb)

    pos2 = posemb.reshape(rows, LANES)
    x3 = x.reshape(B, rows, LANES)

    BB = 8
    out = pl.pallas_call(
        _add_body,
        grid=(B // BB,),
        in_specs=[
            pl.BlockSpec((BB, rows, LANES), lambda i: (i, 0, 0)),
            pl.BlockSpec((rows, LANES), lambda i: (0, 0)),
        ],
        out_specs=pl.BlockSpec((BB, rows, LANES), lambda i: (i, 0, 0)),
        out_shape=jax.ShapeDtypeStruct((B, rows, LANES), x.dtype),
    )(x3, pos2)
    return out.reshape(B, H, W, D)
